# Initial kernel scaffold; baseline (speedup 1.0000x reference)
import jax
import jax.numpy as jnp
from jax.experimental import pallas as pl


def kernel(x_real, x_imag, edge_index, weight, bias):
    raise NotImplementedError("write your pallas kernel here")


# speedup vs baseline: 1.1637x; 1.1637x over previous
import functools
import math

import jax
import jax.numpy as jnp
from jax import lax
from jax.experimental import pallas as pl
from jax.experimental.pallas import tpu as pltpu
from jax.experimental.pallas import tpu_sc as plsc

N_NODES = 10000
Q = 0.25
NW = 32
NPW = 320
NPAD = NW * NPW
CHUNK = 64
FP = 128
NPANEL = 4


def _make_sc_propagate():
    mesh = plsc.VectorSubcoreMesh(core_axis_name="c", subcore_axis_name="s")
    out_type = jax.ShapeDtypeStruct((NPAD * FP,), jnp.float32)
    scratch = [
        pltpu.VMEM((CHUNK,), jnp.int32),
        pltpu.VMEM((CHUNK, FP), jnp.float32),
        pltpu.VMEM((NPW * FP,), jnp.float32),
        pltpu.VMEM((CHUNK,), jnp.int32),
        pltpu.VMEM((CHUNK,), jnp.float32),
        pltpu.VMEM((CHUNK,), jnp.float32),
        pltpu.VMEM((64,), jnp.int32),
        pltpu.SemaphoreType.DMA,
    ]

    @functools.partial(
        pl.kernel, mesh=mesh, out_type=out_type, scratch_types=scratch,
        compiler_params=pltpu.CompilerParams(needs_layout_passes=False))
    def run(tbl, csrc, edst, ewr, ewi, eoff, out,
            idx_v, rows_v, acc, dst_v, wr_v, wi_v, eo_v, sem):
        sid = lax.axis_index("s")
        wid = sid * 2 + lax.axis_index("c")
        lane = lax.broadcasted_iota(jnp.int32, (16,), 0)
        pltpu.sync_copy(eoff, eo_v)
        e_lo = jnp.max(eo_v[pl.ds(wid, 16)])
        e_hi = jnp.max(eo_v[pl.ds(wid + 1, 16)])
        base = wid * NPW
        cs0 = e_lo & ~7
        n_chunks = (e_hi - cs0 + CHUNK - 1) // CHUNK
        zeros16 = jnp.zeros((16,), jnp.float32)

        def zero_body(i, _):
            acc[pl.ds(i * 16, 16)] = zeros16
            return i0
        i0 = jnp.int32(0)
        lax.fori_loop(i0, jnp.int32(NPW * FP // 16), zero_body, i0)

        def chunk_body(k, _):
            cs = pl.multiple_of(cs0 + k * CHUNK, 8)
            pltpu.sync_copy(csrc.at[pl.ds(cs, CHUNK)], idx_v)
            pltpu.sync_copy(edst.at[pl.ds(cs, CHUNK)], dst_v)
            pltpu.sync_copy(ewr.at[pl.ds(cs, CHUNK)], wr_v)
            pltpu.sync_copy(ewi.at[pl.ds(cs, CHUNK)], wi_v)
            pltpu.async_copy(tbl.at[idx_v], rows_v, sem).wait()

            def edge_body(e, _):
                ob = e - cs
                grp = ob & ~15
                ln16 = jnp.full((16,), ob - grp, jnp.int32)
                dvec = dst_v[pl.ds(grp, 16)]
                wrv = wr_v[pl.ds(grp, 16)]
                wiv = wi_v[pl.ds(grp, 16)]
                dl = dvec.at[ln16].get(mode="promise_in_bounds")
                w_r = wrv.at[ln16].get(mode="promise_in_bounds")
                w_i = wiv.at[ln16].get(mode="promise_in_bounds")
                addr0 = (dl - base) * FP + lane
                for j in range(FP // 16):
                    w = w_r if j < FP // 32 else w_i
                    v = w * rows_v[ob, pl.ds(16 * j, 16)]
                    plsc.addupdate_scatter(acc, [addr0 + 16 * j], v)
                return i0
            lax.fori_loop(jnp.maximum(e_lo, cs),
                          jnp.minimum(e_hi, cs + CHUNK), edge_body, i0)
            return i0
        lax.fori_loop(i0, n_chunks, chunk_body, i0)
        pltpu.sync_copy(acc, out.at[pl.ds(base * FP, NPW * FP)])

    return run


def _tc_combine(xr, xi, a_p, b_p, w_stack, bias2d):
    BLK = 400
    grid = (N_NODES // BLK,)

    def body(xr_b, xi_b, a0, a1, a2, a3, b0, b1, b2, b3, ws, bb,
             out_r, out_i):
        def half(a, lo):
            return a[:, lo:lo + 64]
        u = jnp.concatenate([
            xr_b[...] - xi_b[...],
            half(a0, 0) - half(a2, 64), half(a1, 0) - half(a3, 64),
            half(b0, 0) - half(b2, 64), half(b1, 0) - half(b3, 64),
        ], axis=1)
        v = jnp.concatenate([
            xr_b[...] + xi_b[...],
            half(a0, 64) + half(a2, 0), half(a1, 64) + half(a3, 0),
            half(b0, 64) + half(b2, 0), half(b1, 64) + half(b3, 0),
        ], axis=1)
        out_r[...] = jnp.dot(u, ws[...],
                             preferred_element_type=jnp.float32) + bb[...]
        out_i[...] = jnp.dot(v, ws[...],
                             preferred_element_type=jnp.float32) + bb[...]

    row_spec = pl.BlockSpec((BLK, 128), lambda i: (i, i - i))
    full_spec = pl.BlockSpec((384, 128), lambda i: (i - i, i - i))
    bias_spec = pl.BlockSpec((1, 128), lambda i: (i - i, i - i))
    return pl.pallas_call(
        body,
        grid=grid,
        in_specs=[row_spec, row_spec] + [row_spec] * 8
                 + [full_spec, bias_spec],
        out_specs=[row_spec, row_spec],
        out_shape=[jax.ShapeDtypeStruct((N_NODES, 128), jnp.float32),
                   jax.ShapeDtypeStruct((N_NODES, 128), jnp.float32)],
    )(xr, xi, *a_p, *b_p, w_stack, bias2d)


def kernel(x_real, x_imag, edge_index, weight, bias):
    num_nodes = x_real.shape[0]
    row = edge_index[0].astype(jnp.int32)
    col = edge_index[1].astype(jnp.int32)
    w = (row != col).astype(jnp.float32)

    key = jnp.concatenate([row * num_nodes + col, col * num_nodes + row])
    sym = jnp.concatenate([w, w])
    th = jnp.concatenate([w, -w])
    order = jnp.argsort(key)
    ks = key[order]
    sym_s = sym[order]
    th_s = th[order]
    m = ks.shape[0]
    is_start = jnp.concatenate(
        [jnp.ones((1,), dtype=bool), ks[1:] != ks[:-1]])
    seg_id = jnp.cumsum(is_start.astype(jnp.int32)) - 1
    sym_tot = jax.ops.segment_sum(sym_s, seg_id, num_segments=m)
    th_tot = jax.ops.segment_sum(th_s, seg_id, num_segments=m)
    sym_c = jnp.where(is_start, sym_tot[seg_id], jnp.float32(0.0))
    th_c = th_tot[seg_id]
    r = ks // num_nodes
    c = ks % num_nodes
    a_sym = sym_c * 0.5
    theta_w = (2.0 * math.pi * Q) * th_c
    deg = jnp.zeros((num_nodes,), jnp.float32).at[r].add(a_sym)
    dinv = jnp.where(deg > 0, deg ** -0.5, jnp.float32(0.0))
    aw = dinv[r] * a_sym * dinv[c]
    wr = -aw * jnp.cos(theta_w)
    wi = -aw * jnp.sin(theta_w)

    PAD = CHUNK + 8
    zi = jnp.zeros((PAD,), jnp.int32)
    zf = jnp.zeros((PAD,), jnp.float32)
    c_pad = jnp.concatenate([c, zi])
    dst_pad = jnp.concatenate([r, zi])
    wr_pad = jnp.concatenate([wr, zf])
    wi_pad = jnp.concatenate([wi, zf])
    bounds = jnp.arange(33, dtype=jnp.int32) * NPW
    eo = jnp.searchsorted(r, bounds, side="left").astype(jnp.int32)
    eo = jnp.concatenate([jnp.zeros((15,), jnp.int32), eo,
                          jnp.full((16,), m, jnp.int32)])

    x_cat = jnp.concatenate([x_real, x_imag], axis=1)
    x_cat = jnp.pad(x_cat, ((0, NPAD - num_nodes), (0, 0)))
    t1 = [jnp.concatenate([x_cat[:, 64 * p:64 * (p + 1)]] * 2, axis=1)
          for p in range(NPANEL)]

    prop = _make_sc_propagate()
    edge_args = (c_pad, dst_pad, wr_pad, wi_pad, eo)
    a_p = [prop(t, *edge_args).reshape(NPAD, FP) for t in t1]
    b_p = [prop(t, *edge_args).reshape(NPAD, FP) for t in a_p]

    w_stack = jnp.concatenate(
        [weight[0] - weight[2], weight[1], 2.0 * weight[2]], axis=0)
    out_r, out_i = _tc_combine(x_real, x_imag, a_p, b_p, w_stack,
                               bias.reshape(1, 128))
    return out_r, out_i


# speedup vs baseline: 1.3955x; 1.1992x over previous
import functools
import math

import jax
import jax.numpy as jnp
from jax import lax
from jax.experimental import pallas as pl
from jax.experimental.pallas import tpu as pltpu
from jax.experimental.pallas import tpu_sc as plsc

N_NODES = 10000
Q = 0.25
NW = 32
NPW = 320
NPAD = NW * NPW
CHUNK = 256
FP = 128
NPANEL = 4


def _make_sc_propagate():
    mesh = plsc.VectorSubcoreMesh(core_axis_name="c", subcore_axis_name="s")
    out_type = jax.ShapeDtypeStruct((NPAD * FP,), jnp.float32)
    scratch = [
        pltpu.VMEM((CHUNK,), jnp.int32),
        pltpu.VMEM((CHUNK, FP), jnp.float32),
        pltpu.VMEM((NPW * FP,), jnp.float32),
        pltpu.VMEM((CHUNK * 3,), jnp.float32),
        pltpu.VMEM((64,), jnp.int32),
        pltpu.SemaphoreType.DMA,
    ]

    @functools.partial(
        pl.kernel, mesh=mesh, out_type=out_type, scratch_types=scratch,
        compiler_params=pltpu.CompilerParams(needs_layout_passes=False))
    def run(tbl, csrc, emeta, eoff, out,
            idx_v, rows_v, acc, meta_v, eo_v, sem):
        sid = lax.axis_index("s")
        wid = sid * 2 + lax.axis_index("c")
        lane = lax.broadcasted_iota(jnp.int32, (16,), 0)
        pltpu.sync_copy(eoff, eo_v)
        e_lo = jnp.max(eo_v[pl.ds(wid, 16)])
        e_hi = jnp.max(eo_v[pl.ds(wid + 1, 16)])
        base = wid * NPW
        cs0 = e_lo & ~15
        n_chunks = (e_hi - cs0 + CHUNK - 1) // CHUNK
        zeros16 = jnp.zeros((16,), jnp.float32)
        i0 = jnp.int32(0)

        def zero_body(i, _):
            acc[pl.ds(i * 16, 16)] = zeros16
            return i0
        lax.fori_loop(i0, jnp.int32(NPW * FP // 16), zero_body, i0)

        def chunk_body(k, _):
            cs = pl.multiple_of(cs0 + k * CHUNK, 16)
            pltpu.sync_copy(csrc.at[pl.ds(cs, CHUNK)], idx_v)
            pltpu.sync_copy(emeta.at[pl.ds(cs * 3, CHUNK * 3)], meta_v)
            pltpu.async_copy(tbl.at[idx_v], rows_v, sem).wait()

            def group_body(g, _):
                goff = g * 48
                ebase = cs + g * 16
                gidx = ebase + lane
                validf = jnp.where(
                    jnp.logical_and(gidx >= e_lo, gidx < e_hi),
                    jnp.float32(1.0), jnp.float32(0.0))
                dvec = meta_v[pl.ds(goff, 16)].astype(jnp.int32)
                wrv = meta_v[pl.ds(goff + 16, 16)] * validf
                wiv = meta_v[pl.ds(goff + 32, 16)] * validf
                dl16 = jnp.clip(dvec - base, i0, jnp.int32(NPW - 1))
                for l in range(16):
                    ln = jnp.full((16,), l, jnp.int32)
                    w_r = wrv.at[ln].get(mode="promise_in_bounds")
                    w_i = wiv.at[ln].get(mode="promise_in_bounds")
                    dlv = dl16.at[ln].get(mode="promise_in_bounds")
                    addr0 = dlv * FP + lane
                    eoffr = g * 16 + l
                    for j in range(FP // 16):
                        w = w_r if j < FP // 32 else w_i
                        v = w * rows_v[eoffr, pl.ds(16 * j, 16)]
                        plsc.addupdate_scatter(acc, [addr0 + 16 * j], v)
                return i0
            lax.fori_loop(i0, jnp.int32(CHUNK // 16) + (e_lo - e_lo),
                          group_body, i0)
            return i0
        lax.fori_loop(i0, n_chunks, chunk_body, i0)
        pltpu.sync_copy(acc, out.at[pl.ds(base * FP, NPW * FP)])

    return run


def _tc_combine(xr, xi, a_p, b_p, w_stack, bias2d):
    BLK = 400
    grid = (N_NODES // BLK,)

    def body(xr_b, xi_b, a0, a1, a2, a3, b0, b1, b2, b3, ws, bb,
             out_r, out_i):
        def half(a, lo):
            return a[:, lo:lo + 64]
        u = jnp.concatenate([
            xr_b[...] - xi_b[...],
            half(a0, 0) - half(a2, 64), half(a1, 0) - half(a3, 64),
            half(b0, 0) - half(b2, 64), half(b1, 0) - half(b3, 64),
        ], axis=1)
        v = jnp.concatenate([
            xr_b[...] + xi_b[...],
            half(a0, 64) + half(a2, 0), half(a1, 64) + half(a3, 0),
            half(b0, 64) + half(b2, 0), half(b1, 64) + half(b3, 0),
        ], axis=1)
        out_r[...] = jnp.dot(u, ws[...],
                             preferred_element_type=jnp.float32) + bb[...]
        out_i[...] = jnp.dot(v, ws[...],
                             preferred_element_type=jnp.float32) + bb[...]

    row_spec = pl.BlockSpec((BLK, 128), lambda i: (i, i - i))
    full_spec = pl.BlockSpec((384, 128), lambda i: (i - i, i - i))
    bias_spec = pl.BlockSpec((1, 128), lambda i: (i - i, i - i))
    return pl.pallas_call(
        body,
        grid=grid,
        in_specs=[row_spec, row_spec] + [row_spec] * 8
                 + [full_spec, bias_spec],
        out_specs=[row_spec, row_spec],
        out_shape=[jax.ShapeDtypeStruct((N_NODES, 128), jnp.float32),
                   jax.ShapeDtypeStruct((N_NODES, 128), jnp.float32)],
    )(xr, xi, *a_p, *b_p, w_stack, bias2d)


def kernel(x_real, x_imag, edge_index, weight, bias):
    num_nodes = x_real.shape[0]
    row = edge_index[0].astype(jnp.int32)
    col = edge_index[1].astype(jnp.int32)
    w = (row != col).astype(jnp.float32)

    key = jnp.concatenate([row * num_nodes + col, col * num_nodes + row])
    sym = jnp.concatenate([w, w])
    th = jnp.concatenate([w, -w])
    order = jnp.argsort(key)
    ks = key[order]
    sym_s = sym[order]
    th_s = th[order]
    m = ks.shape[0]
    is_start = jnp.concatenate(
        [jnp.ones((1,), dtype=bool), ks[1:] != ks[:-1]])
    seg_id = jnp.cumsum(is_start.astype(jnp.int32)) - 1
    sym_tot = jax.ops.segment_sum(sym_s, seg_id, num_segments=m)
    th_tot = jax.ops.segment_sum(th_s, seg_id, num_segments=m)
    sym_c = jnp.where(is_start, sym_tot[seg_id], jnp.float32(0.0))
    th_c = th_tot[seg_id]
    r = ks // num_nodes
    c = ks % num_nodes
    a_sym = sym_c * 0.5
    theta_w = (2.0 * math.pi * Q) * th_c
    deg = jnp.zeros((num_nodes,), jnp.float32).at[r].add(a_sym)
    dinv = jnp.where(deg > 0, deg ** -0.5, jnp.float32(0.0))
    aw = dinv[r] * a_sym * dinv[c]
    wr = -aw * jnp.cos(theta_w)
    wi = -aw * jnp.sin(theta_w)

    PAD = CHUNK + 16
    zi = jnp.zeros((PAD,), jnp.int32)
    zf = jnp.zeros((PAD,), jnp.float32)
    c_pad = jnp.concatenate([c, zi])
    dst_pad = jnp.concatenate([r, zi]).astype(jnp.float32)
    wr_pad = jnp.concatenate([wr, zf])
    wi_pad = jnp.concatenate([wi, zf])
    emeta = jnp.stack([dst_pad.reshape(-1, 16), wr_pad.reshape(-1, 16),
                       wi_pad.reshape(-1, 16)], axis=1).reshape(-1)
    bounds = jnp.arange(33, dtype=jnp.int32) * NPW
    eo = jnp.searchsorted(r, bounds, side="left").astype(jnp.int32)
    eo = jnp.concatenate([jnp.zeros((15,), jnp.int32), eo,
                          jnp.full((16,), m, jnp.int32)])

    x_cat = jnp.concatenate([x_real, x_imag], axis=1)
    x_cat = jnp.pad(x_cat, ((0, NPAD - num_nodes), (0, 0)))
    t1 = [jnp.concatenate([x_cat[:, 64 * p:64 * (p + 1)]] * 2, axis=1)
          for p in range(NPANEL)]

    prop = _make_sc_propagate()
    edge_args = (c_pad, emeta, eo)
    a_p = [prop(t, *edge_args).reshape(NPAD, FP) for t in t1]
    b_p = [prop(t, *edge_args).reshape(NPAD, FP) for t in a_p]

    w_stack = jnp.concatenate(
        [weight[0] - weight[2], weight[1], 2.0 * weight[2]], axis=0)
    out_r, out_i = _tc_combine(x_real, x_imag, a_p, b_p, w_stack,
                               bias.reshape(1, 128))
    return out_r, out_i


# speedup vs baseline: 1.4488x; 1.0382x over previous
import functools
import math

import jax
import jax.numpy as jnp
from jax import lax
from jax.experimental import pallas as pl
from jax.experimental.pallas import tpu as pltpu
from jax.experimental.pallas import tpu_sc as plsc

N_NODES = 10000
Q = 0.25
NW = 32
NPW = 320
NPAD = NW * NPW
CHUNK = 256
FP = 128
NPANEL = 4


def _make_sc_propagate():
    mesh = plsc.VectorSubcoreMesh(core_axis_name="c", subcore_axis_name="s")
    out_type = jax.ShapeDtypeStruct((NPAD * FP,), jnp.float32)
    scratch = [
        pltpu.VMEM((CHUNK,), jnp.int32),
        pltpu.VMEM((CHUNK, FP), jnp.float32),
        pltpu.VMEM((NPW * FP,), jnp.float32),
        pltpu.VMEM((CHUNK * 3,), jnp.float32),
        pltpu.VMEM((64,), jnp.int32),
        pltpu.SemaphoreType.DMA,
    ]

    @functools.partial(
        pl.kernel, mesh=mesh, out_type=out_type, scratch_types=scratch,
        compiler_params=pltpu.CompilerParams(needs_layout_passes=False))
    def run(tbl, csrc, emeta, eoff, out,
            idx_v, rows_v, acc, meta_v, eo_v, sem):
        sid = lax.axis_index("s")
        wid = sid * 2 + lax.axis_index("c")
        lane = lax.broadcasted_iota(jnp.int32, (16,), 0)
        pltpu.sync_copy(eoff, eo_v)
        e_lo = jnp.max(eo_v[pl.ds(wid, 16)])
        e_hi = jnp.max(eo_v[pl.ds(wid + 1, 16)])
        base = wid * NPW
        cs0 = e_lo & ~15
        n_chunks = (e_hi - cs0 + CHUNK - 1) // CHUNK
        zeros16 = jnp.zeros((16,), jnp.float32)
        i0 = jnp.int32(0)

        def zero_body(i, _):
            acc[pl.ds(i * 16, 16)] = zeros16
            return i0
        lax.fori_loop(i0, jnp.int32(NPW * FP // 16), zero_body, i0)

        def chunk_body(k, _):
            cs = pl.multiple_of(cs0 + k * CHUNK, 16)
            pltpu.sync_copy(csrc.at[pl.ds(cs, CHUNK)], idx_v)
            pltpu.sync_copy(emeta.at[pl.ds(cs * 3, CHUNK * 3)], meta_v)
            pltpu.async_copy(tbl.at[idx_v], rows_v, sem).wait()

            def group_body(g, _):
                goff = g * 48
                ebase = cs + g * 16
                gidx = ebase + lane
                validf = jnp.where(
                    jnp.logical_and(gidx >= e_lo, gidx < e_hi),
                    jnp.float32(1.0), jnp.float32(0.0))
                dvec = meta_v[pl.ds(goff, 16)].astype(jnp.int32)
                wrv = meta_v[pl.ds(goff + 16, 16)] * validf
                wiv = meta_v[pl.ds(goff + 32, 16)] * validf
                dl16 = jnp.clip(dvec - base, i0, jnp.int32(NPW - 1))
                for l in range(16):
                    ln = jnp.full((16,), l, jnp.int32)
                    w_r = wrv.at[ln].get(mode="promise_in_bounds")
                    w_i = wiv.at[ln].get(mode="promise_in_bounds")
                    dlv = dl16.at[ln].get(mode="promise_in_bounds")
                    addr0 = dlv * FP + lane
                    eoffr = g * 16 + l
                    for j in range(FP // 16):
                        w = w_r if j < FP // 32 else w_i
                        v = w * rows_v[eoffr, pl.ds(16 * j, 16)]
                        plsc.addupdate_scatter(acc, [addr0 + 16 * j], v)
                return i0
            lax.fori_loop(i0, jnp.int32(CHUNK // 16) + (e_lo - e_lo),
                          group_body, i0)
            return i0
        lax.fori_loop(i0, n_chunks, chunk_body, i0)
        pltpu.sync_copy(acc, out.at[pl.ds(base * FP, NPW * FP)])

    return run


def _tc_combine(xr, xi, a_p, b_p, w_stack, bias2d):
    BLK = 400
    grid = (N_NODES // BLK,)

    def body(xr_b, xi_b, a0, a1, a2, a3, b0, b1, b2, b3, ws, bb,
             out_r, out_i):
        def half(a, lo):
            return a[:, lo:lo + 64]
        u = jnp.concatenate([
            xr_b[...] - xi_b[...],
            half(a0, 0) - half(a2, 64), half(a1, 0) - half(a3, 64),
            half(b0, 0) - half(b2, 64), half(b1, 0) - half(b3, 64),
        ], axis=1)
        v = jnp.concatenate([
            xr_b[...] + xi_b[...],
            half(a0, 64) + half(a2, 0), half(a1, 64) + half(a3, 0),
            half(b0, 64) + half(b2, 0), half(b1, 64) + half(b3, 0),
        ], axis=1)
        out_r[...] = jnp.dot(u, ws[...],
                             preferred_element_type=jnp.float32) + bb[...]
        out_i[...] = jnp.dot(v, ws[...],
                             preferred_element_type=jnp.float32) + bb[...]

    row_spec = pl.BlockSpec((BLK, 128), lambda i: (i, i - i))
    full_spec = pl.BlockSpec((384, 128), lambda i: (i - i, i - i))
    bias_spec = pl.BlockSpec((1, 128), lambda i: (i - i, i - i))
    return pl.pallas_call(
        body,
        grid=grid,
        in_specs=[row_spec, row_spec] + [row_spec] * 8
                 + [full_spec, bias_spec],
        out_specs=[row_spec, row_spec],
        out_shape=[jax.ShapeDtypeStruct((N_NODES, 128), jnp.float32),
                   jax.ShapeDtypeStruct((N_NODES, 128), jnp.float32)],
    )(xr, xi, *a_p, *b_p, w_stack, bias2d)


def kernel(x_real, x_imag, edge_index, weight, bias):
    num_nodes = x_real.shape[0]
    row = edge_index[0].astype(jnp.int32)
    col = edge_index[1].astype(jnp.int32)
    w = (row != col).astype(jnp.float32)

    key = jnp.concatenate([row * num_nodes + col, col * num_nodes + row])
    sym = jnp.concatenate([w, w])
    th = jnp.concatenate([w, -w])
    ks, sym_s, th_s = lax.sort((key, sym, th), num_keys=1)
    m = ks.shape[0]
    idx = jnp.arange(m, dtype=jnp.int32)
    is_start = jnp.concatenate(
        [jnp.ones((1,), dtype=bool), ks[1:] != ks[:-1]])
    start_idx = jnp.where(is_start, idx, jnp.int32(m))
    nse = jnp.concatenate([
        lax.associative_scan(jnp.minimum, start_idx[1:], reverse=True),
        jnp.full((1,), m, jnp.int32)])
    csym = jnp.cumsum(sym_s)
    cth = jnp.cumsum(th_s)
    last = nse - 1
    sym_tot = csym[last] - csym[idx] + sym_s
    th_tot = cth[last] - cth[idx] + th_s
    r = ks // num_nodes
    c = ks % num_nodes
    a_sym = jnp.where(is_start, sym_tot * 0.5, jnp.float32(0.0))
    theta_w = (2.0 * math.pi * Q) * th_tot
    casym = jnp.concatenate([jnp.zeros((1,), jnp.float32),
                             jnp.cumsum(a_sym)])
    nb = jnp.searchsorted(r, jnp.arange(num_nodes + 1, dtype=jnp.int32),
                          side="left")
    deg = casym[nb[1:]] - casym[nb[:-1]]
    dinv = jnp.where(deg > 0, deg ** -0.5, jnp.float32(0.0))
    aw = dinv[r] * a_sym * dinv[c]
    wr = -aw * jnp.cos(theta_w)
    wi = -aw * jnp.sin(theta_w)

    PAD = CHUNK + 16
    zi = jnp.zeros((PAD,), jnp.int32)
    zf = jnp.zeros((PAD,), jnp.float32)
    c_pad = jnp.concatenate([c, zi])
    dst_pad = jnp.concatenate([r, zi]).astype(jnp.float32)
    wr_pad = jnp.concatenate([wr, zf])
    wi_pad = jnp.concatenate([wi, zf])
    emeta = jnp.stack([dst_pad.reshape(-1, 16), wr_pad.reshape(-1, 16),
                       wi_pad.reshape(-1, 16)], axis=1).reshape(-1)
    bounds = jnp.arange(33, dtype=jnp.int32) * NPW
    eo = jnp.searchsorted(r, bounds, side="left").astype(jnp.int32)
    eo = jnp.concatenate([jnp.zeros((15,), jnp.int32), eo,
                          jnp.full((16,), m, jnp.int32)])

    x_cat = jnp.concatenate([x_real, x_imag], axis=1)
    x_cat = jnp.pad(x_cat, ((0, NPAD - num_nodes), (0, 0)))
    t1 = [jnp.concatenate([x_cat[:, 64 * p:64 * (p + 1)]] * 2, axis=1)
          for p in range(NPANEL)]

    prop = _make_sc_propagate()
    edge_args = (c_pad, emeta, eo)
    a_p = [prop(t, *edge_args).reshape(NPAD, FP) for t in t1]
    b_p = [prop(t, *edge_args).reshape(NPAD, FP) for t in a_p]

    w_stack = jnp.concatenate(
        [weight[0] - weight[2], weight[1], 2.0 * weight[2]], axis=0)
    out_r, out_i = _tc_combine(x_real, x_imag, a_p, b_p, w_stack,
                               bias.reshape(1, 128))
    return out_r, out_i
